# Initial kernel scaffold; baseline (speedup 1.0000x reference)
import jax
import jax.numpy as jnp
from jax.experimental import pallas as pl


def kernel(X, A, b):
    raise NotImplementedError("write your pallas kernel here")


# speedup vs baseline: 2.7956x; 2.7956x over previous
import jax
import jax.numpy as jnp
from jax import lax
from jax.experimental import pallas as pl

N = 16384
D = 128
BLOCK_ROWS = 1024


def _fused_kernel(x_ref, a_ref, b_ref, out_ref):
    x = x_ref[...]
    a = a_ref[...]
    bias = b_ref[...]
    yt = lax.dot_general(a, x, (((1,), (1,)), ((), ())),
                         preferred_element_type=jnp.float32) + bias
    xt = lax.transpose(x, (1, 0))
    maskt = xt != 0
    mv = jnp.where(maskt, yt, 0.0)
    def chunk(lo):
        p = mv[lo:lo + 8, :]
        for v in range(1, 8):
            p = p + mv[lo + 8 * v:lo + 8 * v + 8, :]
        q = p[0:4, :] + p[4:8, :]
        r2 = q[0:2, :] + q[2:4, :]
        return r2[0:1, :] + r2[1:2, :]

    s = chunk(0) + chunk(64)
    out_t = jnp.where(maskt, yt / s, 0.0)
    out_ref[...] = lax.transpose(out_t, (1, 0))


def kernel(X, A, b):
    b2 = b.reshape(D, 1)
    return pl.pallas_call(
        _fused_kernel,
        grid=(N // BLOCK_ROWS,),
        in_specs=[
            pl.BlockSpec((BLOCK_ROWS, D), lambda i: (i, 0)),
            pl.BlockSpec((D, D), lambda i: (0, 0)),
            pl.BlockSpec((D, 1), lambda i: (0, 0)),
        ],
        out_specs=pl.BlockSpec((BLOCK_ROWS, D), lambda i: (i, 0)),
        out_shape=jax.ShapeDtypeStruct((N, D), jnp.float32),
    )(X, A, b2)


# speedup vs baseline: 3.6807x; 1.3166x over previous
import jax
import jax.numpy as jnp
from jax import lax
from jax.experimental import pallas as pl

N = 16384
D = 128
BLOCK_ROWS = 2048


def _fused_kernel(x_ref, a_ref, b_ref, out_ref):
    x = x_ref[...]
    a = a_ref[...]
    bias = b_ref[...]
    yt = lax.dot_general(a, x, (((1,), (1,)), ((), ())),
                         preferred_element_type=jnp.float32) + bias
    xt = lax.transpose(x, (1, 0))
    maskt = xt != 0
    mv = jnp.where(maskt, yt, 0.0)
    def chunk(lo):
        p = mv[lo:lo + 8, :]
        for v in range(1, 8):
            p = p + mv[lo + 8 * v:lo + 8 * v + 8, :]
        q = p[0:4, :] + p[4:8, :]
        r2 = q[0:2, :] + q[2:4, :]
        return r2[0:1, :] + r2[1:2, :]

    s = chunk(0) + chunk(64)
    out_t = jnp.where(maskt, yt / s, 0.0)
    out_ref[...] = lax.transpose(out_t, (1, 0))


def kernel(X, A, b):
    b2 = b.reshape(D, 1)
    return pl.pallas_call(
        _fused_kernel,
        grid=(N // BLOCK_ROWS,),
        in_specs=[
            pl.BlockSpec((BLOCK_ROWS, D), lambda i: (i, 0)),
            pl.BlockSpec((D, D), lambda i: (0, 0)),
            pl.BlockSpec((D, 1), lambda i: (0, 0)),
        ],
        out_specs=pl.BlockSpec((BLOCK_ROWS, D), lambda i: (i, 0)),
        out_shape=jax.ShapeDtypeStruct((N, D), jnp.float32),
    )(X, A, b2)


# speedup vs baseline: 4.3306x; 1.1766x over previous
import jax
import jax.numpy as jnp
from jax import lax
from jax.experimental import pallas as pl

N = 16384
D = 128
BLOCK_ROWS = 4096


def _fused_kernel(x_ref, a_ref, b_ref, out_ref):
    x = x_ref[...]
    a = a_ref[...]
    bias = b_ref[...]
    yt = lax.dot_general(a, x, (((1,), (1,)), ((), ())),
                         preferred_element_type=jnp.float32) + bias
    xt = lax.transpose(x, (1, 0))
    maskt = xt != 0
    mv = jnp.where(maskt, yt, 0.0)
    def chunk(lo):
        p = mv[lo:lo + 8, :]
        for v in range(1, 8):
            p = p + mv[lo + 8 * v:lo + 8 * v + 8, :]
        q = p[0:4, :] + p[4:8, :]
        r2 = q[0:2, :] + q[2:4, :]
        return r2[0:1, :] + r2[1:2, :]

    s = chunk(0) + chunk(64)
    out_t = jnp.where(maskt, yt / s, 0.0)
    out_ref[...] = lax.transpose(out_t, (1, 0))


def kernel(X, A, b):
    b2 = b.reshape(D, 1)
    return pl.pallas_call(
        _fused_kernel,
        grid=(N // BLOCK_ROWS,),
        in_specs=[
            pl.BlockSpec((BLOCK_ROWS, D), lambda i: (i, 0)),
            pl.BlockSpec((D, D), lambda i: (0, 0)),
            pl.BlockSpec((D, 1), lambda i: (0, 0)),
        ],
        out_specs=pl.BlockSpec((BLOCK_ROWS, D), lambda i: (i, 0)),
        out_shape=jax.ShapeDtypeStruct((N, D), jnp.float32),
    )(X, A, b2)


# speedup vs baseline: 4.4356x; 1.0243x over previous
import jax
import jax.numpy as jnp
from jax import lax
from jax.experimental import pallas as pl

N = 16384
D = 128
BLOCK_ROWS = 8192


def _fused_kernel(x_ref, a_ref, b_ref, out_ref):
    x = x_ref[...]
    a = a_ref[...]
    bias = b_ref[...]
    yt = lax.dot_general(a, x, (((1,), (1,)), ((), ())),
                         preferred_element_type=jnp.float32) + bias
    xt = lax.transpose(x, (1, 0))
    maskt = xt != 0
    mv = jnp.where(maskt, yt, 0.0)
    def chunk(lo):
        p = mv[lo:lo + 8, :]
        for v in range(1, 8):
            p = p + mv[lo + 8 * v:lo + 8 * v + 8, :]
        q = p[0:4, :] + p[4:8, :]
        r2 = q[0:2, :] + q[2:4, :]
        return r2[0:1, :] + r2[1:2, :]

    s = chunk(0) + chunk(64)
    out_t = jnp.where(maskt, yt / s, 0.0)
    out_ref[...] = lax.transpose(out_t, (1, 0))


def kernel(X, A, b):
    b2 = b.reshape(D, 1)
    return pl.pallas_call(
        _fused_kernel,
        grid=(N // BLOCK_ROWS,),
        in_specs=[
            pl.BlockSpec((BLOCK_ROWS, D), lambda i: (i, 0)),
            pl.BlockSpec((D, D), lambda i: (0, 0)),
            pl.BlockSpec((D, 1), lambda i: (0, 0)),
        ],
        out_specs=pl.BlockSpec((BLOCK_ROWS, D), lambda i: (i, 0)),
        out_shape=jax.ShapeDtypeStruct((N, D), jnp.float32),
    )(X, A, b2)
